# baseline (speedup 1.0000x reference)
import functools
import jax, jax.numpy as jnp
from jax import lax
from jax.experimental import pallas as pl
from jax.experimental.pallas import tpu as pltpu
from jax.experimental.pallas import tpu_sc as plsc

def kernel(x, tables):
    mesh = plsc.VectorSubcoreMesh(core_axis_name="c", subcore_axis_name="s")
    tref = jax.new_ref(tables)

    @functools.partial(pl.kernel, mesh=mesh,
        out_type=jax.ShapeDtypeStruct((16,), jnp.float32),
        scratch_types=[pltpu.VMEM((32,), jnp.float32)])
    def k(out_hbm, scratch):
        pltpu.sync_copy(tref.at[0, 0], scratch)

    return k()
